# baseline (speedup 1.0000x reference)
import functools

import jax
import jax.numpy as jnp
from jax import lax
from jax.experimental import pallas as pl
from jax.experimental.pallas import tpu as pltpu
from jax.experimental.pallas import tpu_sc as plsc

S = 512
D = 128
T = 2 * S - 1
NC = 2
NS = 16
NW = NC * NS
RW = S // NW
C = 128
NCHUNK = S // C
L = 16


def _sc_body(emb_hbm, x_hbm, out_hbm, xv, buf, idxv, sem):
    wid = lax.axis_index("s") * NC + lax.axis_index("c")

    pltpu.sync_copy(x_hbm, xv)

    iota = lax.iota(jnp.int32, L)
    for r in range(RW):
        i = wid * RW + r
        for c in range(NCHUNK):
            for m in range(C // L):
                idxv[c, pl.ds(m * L, L)] = (i + (S - 1) - c * C - m * L) - iota
        for c in range(NCHUNK):
            pltpu.async_copy(emb_hbm.at[idxv.at[c]], buf, sem).wait()

            def _add(jj, carry, c=c):
                for m in range(D // L):
                    sl = pl.ds(m * L, L)
                    buf[jj, sl] = buf[jj, sl] + xv[c * C + jj, sl]
                return carry

            lax.fori_loop(0, C, _add, 0)

            pltpu.sync_copy(buf, out_hbm.at[pl.ds(i * S + c * C, C)])


_sc_call = functools.partial(
    pl.kernel,
    mesh=plsc.VectorSubcoreMesh(core_axis_name="c", subcore_axis_name="s"),
    out_type=jax.ShapeDtypeStruct((S * S, D), jnp.float32),
    scratch_types=[
        pltpu.VMEM((S, D), jnp.float32),
        pltpu.VMEM((C, D), jnp.float32),
        pltpu.VMEM((NCHUNK, C), jnp.int32),
        pltpu.SemaphoreType.DMA,
    ],
)(_sc_body)


def kernel(x, emb_table):
    out = _sc_call(emb_table, x[0])
    return out.reshape(S, S, D)
